# baseline (speedup 1.0000x reference)
import jax
import jax.numpy as jnp
from jax.experimental import pallas as pl

N = 50000
E = 800000
D = 100


def _conv(x, src, dst, Wq, bq, Wk, bk, Wv, bv, Ws, bs):
    q = x @ Wq + bq
    k = x @ Wk + bk
    v = x @ Wv + bv
    qi = q[dst]
    kj = k[src]
    vj = v[src]
    score = jnp.sum(qi * kj, axis=-1) / jnp.sqrt(jnp.float32(D))
    m = jax.ops.segment_max(score, dst, num_segments=N)
    e = jnp.exp(score - m[dst])
    s = jax.ops.segment_sum(e, dst, num_segments=N)
    alpha = e / s[dst]
    agg = jax.ops.segment_sum(alpha[:, None] * vj, dst, num_segments=N)
    out = agg + x @ Ws + bs
    return out, alpha


def _bn(x, gamma, beta):
    mean = jnp.mean(x, axis=0)
    var = jnp.var(x, axis=0)
    return gamma * (x - mean) * jax.lax.rsqrt(var + 1e-5) + beta


def _block(x0, src, dst, p):
    x, a1 = _conv(x0, src, dst,
                  p["conv1_q_W"], p["conv1_q_b"],
                  p["conv1_k_W"], p["conv1_k_b"],
                  p["conv1_v_W"], p["conv1_v_b"],
                  p["conv1_skip_W"], p["conv1_skip_b"])
    x = jax.nn.relu(_bn(x, p["bn1_gamma"], p["bn1_beta"]))
    x, a2 = _conv(x, src, dst,
                  p["conv2_q_W"], p["conv2_q_b"],
                  p["conv2_k_W"], p["conv2_k_b"],
                  p["conv2_v_W"], p["conv2_v_b"],
                  p["conv2_skip_W"], p["conv2_skip_b"])
    x = _bn(x, p["bn2_gamma"], p["bn2_beta"])
    x = jax.nn.relu(x + x0)
    return x, a1 + a2


def kernel(x, edges, params):
    perm = jnp.argsort(edges[1])
    src = edges[0][perm]
    dst = edges[1][perm]
    x, a0 = _block(x, src, dst, params[0])
    x, a1 = _block(x, src, dst, params[1])
    x, a2 = _block(x, src, dst, params[2])
    att_s = a0 + a1 + a2
    att = jnp.zeros((E,), jnp.float32).at[perm].set(att_s)[:, None]
    return (x, (edges, att))


# speedup vs baseline: 2.3929x; 2.3929x over previous
import functools

import jax
import jax.numpy as jnp
from jax import lax
from jax.experimental import pallas as pl
from jax.experimental.pallas import tpu as pltpu
from jax.experimental.pallas import tpu_sc as plsc

N = 50000
E = 800000
D = 100
DP = 128
NP = 50176
EP = 802816
NSENT = NP - N
NC = 2
NS = 16
NW = NC * NS
ESPAN = EP // NW
C = 112
NCHUNK = ESPAN // C
SUBN = NP // NS
INVSQ = 0.1
RNG = NP // 4
ESPAN2 = EP // NS
NCHUNK2 = ESPAN2 // C
FL = 56
NFL = 784 // FL
BM = 512
GRID = NP // BM

_mesh = plsc.VectorSubcoreMesh(core_axis_name="c", subcore_axis_name="s")
_sc_params = pltpu.CompilerParams(needs_layout_passes=False)


def _p1_body(qp, kp, srce, dste, msh, ev_out, s0_out, s1_out,
             idx_s, idx_d, qrows, krows, mrow, evb, zb, shacc, sem):
    cid = lax.axis_index("c")
    sid = lax.axis_index("s")
    wid = sid * NC + cid

    def zeros16(i, _):
        zb[pl.ds(i * 16, 16)] = jnp.zeros((16,), jnp.float32)
        return 0

    lax.fori_loop(0, SUBN // 16, zeros16, 0)
    pltpu.sync_copy(zb, shacc.at[pl.ds(sid * SUBN, SUBN)])
    plsc.subcore_barrier()

    def chunk(j, _):
        base = wid * ESPAN + j * C
        pltpu.sync_copy(srce.at[pl.ds(base, C)], idx_s)
        pltpu.sync_copy(dste.at[pl.ds(base, C)], idx_d)
        pltpu.async_copy(qp.at[idx_d], qrows, sem).wait()
        pltpu.async_copy(kp.at[idx_s], krows, sem).wait()
        pltpu.async_copy(msh.at[idx_d], mrow, sem).wait()

        def group(g, _):
            lanes = g * 16 + lax.iota(jnp.int32, 16)

            def dim(d, acc):
                dd = jnp.full((16,), d, jnp.int32)
                qv = plsc.load_gather(qrows, [lanes, dd])
                kv = plsc.load_gather(krows, [lanes, dd])
                return acc + qv * kv

            acc = lax.fori_loop(0, D, dim, jnp.zeros((16,), jnp.float32))
            mg = mrow[pl.ds(g * 16, 16)]
            evb[pl.ds(g * 16, 16)] = jnp.exp(acc * INVSQ - mg)
            return 0

        lax.fori_loop(0, C // 16, group, 0)

        pltpu.sync_copy(evb, ev_out.at[pl.ds(base, C)])
        pltpu.sync_copy(evb, shacc.at[idx_d], add=True)
        return 0

    lax.fori_loop(0, NCHUNK, chunk, 0)

    plsc.subcore_barrier()
    pltpu.sync_copy(shacc.at[pl.ds(sid * SUBN, SUBN)], zb)

    @pl.when(cid == 0)
    def _():
        pltpu.sync_copy(zb, s0_out.at[pl.ds(sid * SUBN, SUBN)])

    @pl.when(cid == 1)
    def _():
        pltpu.sync_copy(zb, s1_out.at[pl.ds(sid * SUBN, SUBN)])


@jax.jit
def _p1(qp, kp, srce, dste, msh):
    return pl.kernel(
        _p1_body,
        out_type=[
            jax.ShapeDtypeStruct((EP,), jnp.float32),
            jax.ShapeDtypeStruct((NP,), jnp.float32),
            jax.ShapeDtypeStruct((NP,), jnp.float32),
        ],
        mesh=_mesh,
        compiler_params=_sc_params,
        scratch_types=[
            pltpu.VMEM((C,), jnp.int32),
            pltpu.VMEM((C,), jnp.int32),
            pltpu.VMEM((C, DP), jnp.float32),
            pltpu.VMEM((C, DP), jnp.float32),
            pltpu.VMEM((C,), jnp.float32),
            pltpu.VMEM((C,), jnp.float32),
            pltpu.VMEM((SUBN,), jnp.float32),
            pltpu.VMEM_SHARED((NP,), jnp.float32),
            pltpu.SemaphoreType.DMA,
        ],
    )(qp, kp, srce, dste, msh)


def _p2a_body(ev, dste, s0, s1, att_in, alpha_out, att_out,
              idx_d, evb, attb, s0b, s1b, sem):
    cid = lax.axis_index("c")
    sid = lax.axis_index("s")
    wid = sid * NC + cid

    def chunk(j, _):
        base = wid * ESPAN + j * C
        pltpu.sync_copy(dste.at[pl.ds(base, C)], idx_d)
        pltpu.sync_copy(ev.at[pl.ds(base, C)], evb)
        pltpu.sync_copy(att_in.at[pl.ds(base, C)], attb)
        pltpu.async_copy(s0.at[idx_d], s0b, sem).wait()
        pltpu.async_copy(s1.at[idx_d], s1b, sem).wait()

        def group(g, _):
            sl = pl.ds(g * 16, 16)
            al = evb[sl] / (s0b[sl] + s1b[sl])
            attb[sl] = attb[sl] + al
            evb[sl] = al
            return 0

        lax.fori_loop(0, C // 16, group, 0)
        pltpu.sync_copy(evb, alpha_out.at[pl.ds(base, C)])
        pltpu.sync_copy(attb, att_out.at[pl.ds(base, C)])
        return 0

    lax.fori_loop(0, NCHUNK, chunk, 0)


@jax.jit
def _p2a(ev, dste, s0, s1, att_in):
    return pl.kernel(
        _p2a_body,
        out_type=[
            jax.ShapeDtypeStruct((EP,), jnp.float32),
            jax.ShapeDtypeStruct((EP,), jnp.float32),
        ],
        mesh=_mesh,
        compiler_params=_sc_params,
        scratch_types=[
            pltpu.VMEM((C,), jnp.int32),
            pltpu.VMEM((C,), jnp.float32),
            pltpu.VMEM((C,), jnp.float32),
            pltpu.VMEM((C,), jnp.float32),
            pltpu.VMEM((C,), jnp.float32),
            pltpu.SemaphoreType.DMA,
        ],
    )(ev, dste, s0, s1, att_in)


def _p2b_body(vp, srce, dste, alpha, agg_out,
              idx_s, idx_d, alb, lidx, vrows, flushb, shacc, sem):
    cid = lax.axis_index("c")
    sid = lax.axis_index("s")

    for rp in range(2):
        lo = (cid * 2 + rp) * RNG

        def zf16(i, _):
            r = i // (DP // 16)
            t = i % (DP // 16)
            flushb[r, pl.ds(t * 16, 16)] = jnp.zeros((16,), jnp.float32)
            return 0

        lax.fori_loop(0, FL * (DP // 16), zf16, 0)
        for h in range(NFL):
            pltpu.sync_copy(
                flushb, shacc.at[pl.ds(sid * NFL * FL + h * FL, FL)])
        plsc.subcore_barrier()

        def chunk(j, _):
            base = sid * ESPAN2 + j * C
            pltpu.sync_copy(srce.at[pl.ds(base, C)], idx_s)
            pltpu.sync_copy(dste.at[pl.ds(base, C)], idx_d)
            pltpu.sync_copy(alpha.at[pl.ds(base, C)], alb)
            pltpu.async_copy(vp.at[idx_s], vrows, sem).wait()

            for g in range(C // 16):
                sl = pl.ds(g * 16, 16)
                dg = idx_d[sl]
                inr = (dg >= lo) & (dg < lo + RNG)
                alb[sl] = jnp.where(inr, alb[sl], 0.0)
                lidx[sl] = lax.rem(dg, RNG)

            def edge(e, _):
                ab = plsc.load_gather(alb, [jnp.full((16,), e, jnp.int32)])
                for t in range(DP // 16):
                    sl = pl.ds(t * 16, 16)
                    vrows[e, sl] = ab * vrows[e, sl]
                return 0

            lax.fori_loop(0, C, edge, 0)
            pltpu.sync_copy(vrows, shacc.at[lidx], add=True)
            return 0

        lax.fori_loop(0, NCHUNK2, chunk, 0)
        plsc.subcore_barrier()

        for h in range(NFL):
            roff = sid * NFL * FL + h * FL
            pltpu.sync_copy(shacc.at[pl.ds(roff, FL)], flushb)
            pltpu.sync_copy(flushb, agg_out.at[pl.ds(lo + roff, FL)])
        plsc.subcore_barrier()


@jax.jit
def _p2b(vp, srce, dste, alpha):
    return pl.kernel(
        _p2b_body,
        out_type=jax.ShapeDtypeStruct((NP, DP), jnp.float32),
        mesh=_mesh,
        compiler_params=_sc_params,
        scratch_types=[
            pltpu.VMEM((C,), jnp.int32),
            pltpu.VMEM((C,), jnp.int32),
            pltpu.VMEM((C,), jnp.float32),
            pltpu.VMEM((C,), jnp.int32),
            pltpu.VMEM((C, DP), jnp.float32),
            pltpu.VMEM((FL, DP), jnp.float32),
            pltpu.VMEM_SHARED((RNG, DP), jnp.float32),
            pltpu.SemaphoreType.DMA,
        ],
    )(vp, srce, dste, alpha)


def _dense1_body(x_ref, w_ref, b_ref, q_ref, k_ref, v_ref, s_ref):
    acc = jnp.dot(x_ref[...], w_ref[...],
                  preferred_element_type=jnp.float32)
    acc = acc + b_ref[0:1, :]
    q_ref[...] = acc[:, 0:DP]
    k_ref[...] = acc[:, DP:2 * DP]
    v_ref[...] = acc[:, 2 * DP:3 * DP]
    s_ref[...] = acc[:, 3 * DP:4 * DP]


@jax.jit
def _dense1(xp, w4, b4):
    blk = lambda i: (i, 0)
    full = lambda i: (0, 0)
    return pl.pallas_call(
        _dense1_body,
        grid=(GRID,),
        in_specs=[
            pl.BlockSpec((BM, DP), blk),
            pl.BlockSpec((DP, 4 * DP), full),
            pl.BlockSpec((8, 4 * DP), full),
        ],
        out_specs=[pl.BlockSpec((BM, DP), blk)] * 4,
        out_shape=[jax.ShapeDtypeStruct((NP, DP), jnp.float32)] * 4,
    )(xp, w4, b4)


def _stats_body(agg_ref, skip_ref, y_ref, sum_ref, ssq_ref):
    i = pl.program_id(0)
    rows = i * BM + lax.broadcasted_iota(jnp.int32, (BM, DP), 0)
    y = agg_ref[...] + skip_ref[...]
    y = jnp.where(rows < N, y, 0.0)
    y_ref[...] = y
    ps = jnp.zeros((8, DP), jnp.float32)
    pq = jnp.zeros((8, DP), jnp.float32)
    for j in range(BM // 8):
        blkv = y[j * 8:(j + 1) * 8, :]
        ps = ps + blkv
        pq = pq + blkv * blkv

    @pl.when(i == 0)
    def _():
        sum_ref[...] = ps
        ssq_ref[...] = pq

    @pl.when(i > 0)
    def _():
        sum_ref[...] = sum_ref[...] + ps
        ssq_ref[...] = ssq_ref[...] + pq


@jax.jit
def _stats(agg, skip):
    blk = lambda i: (i, 0)
    acc = lambda i: (0, 0)
    return pl.pallas_call(
        _stats_body,
        grid=(GRID,),
        in_specs=[pl.BlockSpec((BM, DP), blk), pl.BlockSpec((BM, DP), blk)],
        out_specs=[pl.BlockSpec((BM, DP), blk),
                   pl.BlockSpec((8, DP), acc), pl.BlockSpec((8, DP), acc)],
        out_shape=[jax.ShapeDtypeStruct((NP, DP), jnp.float32),
                   jax.ShapeDtypeStruct((8, DP), jnp.float32),
                   jax.ShapeDtypeStruct((8, DP), jnp.float32)],
    )(agg, skip)


def _apply_body(y_ref, sum_ref, ssq_ref, g_ref, b_ref, x0_ref, out_ref,
                *, residual):
    i = pl.program_id(0)
    colsum = jnp.sum(sum_ref[...], axis=0, keepdims=True)
    colssq = jnp.sum(ssq_ref[...], axis=0, keepdims=True)
    mean = colsum / N
    var = colssq / N - mean * mean
    scale = g_ref[0:1, :] * lax.rsqrt(var + 1e-5)
    out = (y_ref[...] - mean) * scale + b_ref[0:1, :]
    if residual:
        out = out + x0_ref[...]
    out = jnp.maximum(out, 0.0)
    rows = i * BM + lax.broadcasted_iota(jnp.int32, (BM, DP), 0)
    out_ref[...] = jnp.where(rows < N, out, 0.0)


@functools.partial(jax.jit, static_argnames=("residual",))
def _apply(y, s1, s2, g, b, x0, residual):
    blk = lambda i: (i, 0)
    full = lambda i: (0, 0)
    return pl.pallas_call(
        functools.partial(_apply_body, residual=residual),
        grid=(GRID,),
        in_specs=[pl.BlockSpec((BM, DP), blk),
                  pl.BlockSpec((8, DP), full), pl.BlockSpec((8, DP), full),
                  pl.BlockSpec((8, DP), full), pl.BlockSpec((8, DP), full),
                  pl.BlockSpec((BM, DP), blk)],
        out_specs=pl.BlockSpec((BM, DP), blk),
        out_shape=jax.ShapeDtypeStruct((NP, DP), jnp.float32),
    )(y, s1, s2, g, b, x0)


def _pad_edges(edges):
    src = edges[0].astype(jnp.int32)
    dst = edges[1].astype(jnp.int32)
    pad = EP - E
    sent = N + (jnp.arange(pad, dtype=jnp.int32) % NSENT)
    return jnp.concatenate([src, sent]), jnp.concatenate([dst, sent])


def _w4(p, pre):
    ws, bs = [], []
    for name in ("q", "k", "v", "skip"):
        W = p[pre + "_" + name + "_W"]
        b = p[pre + "_" + name + "_b"]
        ws.append(jnp.zeros((DP, DP), jnp.float32).at[:D, :D].set(W))
        bs.append(jnp.zeros((DP,), jnp.float32).at[:D].set(b))
    w4 = jnp.concatenate(ws, axis=1)
    b4 = jnp.tile(jnp.concatenate(bs)[None, :], (8, 1))
    return w4, b4


def _tile8(vec):
    v = jnp.zeros((DP,), jnp.float32).at[:D].set(vec)
    return jnp.tile(v[None, :], (8, 1))


def _conv(xp, srce, dste, att_in, p, pre):
    w4, b4 = _w4(p, pre)
    q, k, v, skip = _dense1(xp, w4, b4)
    kmax2 = jnp.max(jnp.sum(k * k, axis=1))
    mshift = (jnp.sum(q * q, axis=1) + kmax2) * (0.5 * INVSQ)
    ev, s0, s1 = _p1(q, k, srce, dste, mshift)
    alpha, att_out = _p2a(ev, dste, s0, s1, att_in)
    agg = _p2b(v, srce, dste, alpha)
    return agg, skip, att_out


def _block(x0, srce, dste, att_in, p):
    agg, skip, att1 = _conv(x0, srce, dste, att_in, p, "conv1")
    y, s1, s2 = _stats(agg, skip)
    x = _apply(y, s1, s2, _tile8(p["bn1_gamma"]), _tile8(p["bn1_beta"]),
               x0, residual=False)
    agg, skip, att2 = _conv(x, srce, dste, att1, p, "conv2")
    y, s1, s2 = _stats(agg, skip)
    x = _apply(y, s1, s2, _tile8(p["bn2_gamma"]), _tile8(p["bn2_beta"]),
               x0, residual=True)
    return x, att2


def kernel(x, edges, params):
    srce, dste = _pad_edges(edges)
    xp = jnp.zeros((NP, DP), jnp.float32).at[:N, :D].set(x)
    att = jnp.zeros((EP,), jnp.float32)
    xp, att = _block(xp, srce, dste, att, params[0])
    xp, att = _block(xp, srce, dste, att, params[1])
    xp, att = _block(xp, srce, dste, att, params[2])
    return (xp[:N, :D], (edges, att[:E, None]))


# speedup vs baseline: 2.7899x; 1.1659x over previous
import functools

import jax
import jax.numpy as jnp
from jax import lax
from jax.experimental import pallas as pl
from jax.experimental.pallas import tpu as pltpu
from jax.experimental.pallas import tpu_sc as plsc

N = 50000
E = 800000
D = 100
DP = 128
NP = 50176
EP = 802816
NSENT = NP - N
NC = 2
NS = 16
NW = NC * NS
ESPAN = EP // NW
C = 112
NCHUNK = ESPAN // C
SUBN = NP // NS
INVSQ = 0.1
RNG = NP // 4
ESPAN2 = EP // NS
C2 = 64
NCHUNK2 = ESPAN2 // C2
FL = 56
NFL = 784 // FL
BM = 512
GRID = NP // BM

_mesh = plsc.VectorSubcoreMesh(core_axis_name="c", subcore_axis_name="s")
_sc_params = pltpu.CompilerParams(needs_layout_passes=False)


def _p1_compute(qrows, krows, mrow, evb):
    def group(g, _):
        lanes = g * 16 + lax.iota(jnp.int32, 16)
        acc = jnp.zeros((16,), jnp.float32)
        for d in range(D):
            dd = jnp.full((16,), d, jnp.int32)
            qv = plsc.load_gather(qrows, [lanes, dd])
            kv = plsc.load_gather(krows, [lanes, dd])
            acc = acc + qv * kv
        mg = mrow[pl.ds(g * 16, 16)]
        evb[pl.ds(g * 16, 16)] = jnp.exp(acc * INVSQ - mg)
        return 0

    lax.fori_loop(0, C // 16, group, 0)


def _p1_body(qp, kp, srce, dste, msh, ev_out, s0_out, s1_out,
             idx_s0, idx_d0, q0, k0, m0, ev0,
             idx_s1, idx_d1, q1, k1, m1, ev1,
             zb, shacc, sem0, sem1):
    cid = lax.axis_index("c")
    sid = lax.axis_index("s")
    wid = sid * NC + cid

    def zeros16(i, _):
        zb[pl.ds(i * 16, 16)] = jnp.zeros((16,), jnp.float32)
        return 0

    lax.fori_loop(0, SUBN // 16, zeros16, 0)
    pltpu.sync_copy(zb, shacc.at[pl.ds(sid * SUBN, SUBN)])
    plsc.subcore_barrier()

    def pair(jj, _):
        b0 = wid * ESPAN + (2 * jj) * C
        b1 = b0 + C
        pltpu.sync_copy(srce.at[pl.ds(b0, C)], idx_s0)
        pltpu.sync_copy(dste.at[pl.ds(b0, C)], idx_d0)
        h0q = pltpu.async_copy(qp.at[idx_d0], q0, sem0)
        h0k = pltpu.async_copy(kp.at[idx_s0], k0, sem0)
        h0m = pltpu.async_copy(msh.at[idx_d0], m0, sem0)
        pltpu.sync_copy(srce.at[pl.ds(b1, C)], idx_s1)
        pltpu.sync_copy(dste.at[pl.ds(b1, C)], idx_d1)
        h1q = pltpu.async_copy(qp.at[idx_d1], q1, sem1)
        h1k = pltpu.async_copy(kp.at[idx_s1], k1, sem1)
        h1m = pltpu.async_copy(msh.at[idx_d1], m1, sem1)
        h0q.wait(); h0k.wait(); h0m.wait()
        _p1_compute(q0, k0, m0, ev0)
        pltpu.sync_copy(ev0, ev_out.at[pl.ds(b0, C)])
        pltpu.sync_copy(ev0, shacc.at[idx_d0], add=True)
        h1q.wait(); h1k.wait(); h1m.wait()
        _p1_compute(q1, k1, m1, ev1)
        pltpu.sync_copy(ev1, ev_out.at[pl.ds(b1, C)])
        pltpu.sync_copy(ev1, shacc.at[idx_d1], add=True)
        return 0

    lax.fori_loop(0, NCHUNK // 2, pair, 0)

    plsc.subcore_barrier()
    pltpu.sync_copy(shacc.at[pl.ds(sid * SUBN, SUBN)], zb)

    @pl.when(cid == 0)
    def _():
        pltpu.sync_copy(zb, s0_out.at[pl.ds(sid * SUBN, SUBN)])

    @pl.when(cid == 1)
    def _():
        pltpu.sync_copy(zb, s1_out.at[pl.ds(sid * SUBN, SUBN)])


@jax.jit
def _p1(qp, kp, srce, dste, msh):
    return pl.kernel(
        _p1_body,
        out_type=[
            jax.ShapeDtypeStruct((EP,), jnp.float32),
            jax.ShapeDtypeStruct((NP,), jnp.float32),
            jax.ShapeDtypeStruct((NP,), jnp.float32),
        ],
        mesh=_mesh,
        compiler_params=_sc_params,
        scratch_types=(
            [pltpu.VMEM((C,), jnp.int32),
             pltpu.VMEM((C,), jnp.int32),
             pltpu.VMEM((C, DP), jnp.float32),
             pltpu.VMEM((C, DP), jnp.float32),
             pltpu.VMEM((C,), jnp.float32),
             pltpu.VMEM((C,), jnp.float32)] * 2
            + [pltpu.VMEM((SUBN,), jnp.float32),
               pltpu.VMEM_SHARED((NP,), jnp.float32),
               pltpu.SemaphoreType.DMA,
               pltpu.SemaphoreType.DMA]
        ),
    )(qp, kp, srce, dste, msh)


def _p2a_body(ev, dste, s0, s1, att_in, alpha_out, att_out,
              idx_d, evb, attb, s0b, s1b, sem):
    cid = lax.axis_index("c")
    sid = lax.axis_index("s")
    wid = sid * NC + cid

    def chunk(j, _):
        base = wid * ESPAN + j * C
        pltpu.sync_copy(dste.at[pl.ds(base, C)], idx_d)
        h0 = pltpu.async_copy(s0.at[idx_d], s0b, sem)
        h1 = pltpu.async_copy(s1.at[idx_d], s1b, sem)
        pltpu.sync_copy(ev.at[pl.ds(base, C)], evb)
        pltpu.sync_copy(att_in.at[pl.ds(base, C)], attb)
        h0.wait()
        h1.wait()

        def group(g, _):
            sl = pl.ds(g * 16, 16)
            al = evb[sl] / (s0b[sl] + s1b[sl])
            attb[sl] = attb[sl] + al
            evb[sl] = al
            return 0

        lax.fori_loop(0, C // 16, group, 0)
        pltpu.sync_copy(evb, alpha_out.at[pl.ds(base, C)])
        pltpu.sync_copy(attb, att_out.at[pl.ds(base, C)])
        return 0

    lax.fori_loop(0, NCHUNK, chunk, 0)


@jax.jit
def _p2a(ev, dste, s0, s1, att_in):
    return pl.kernel(
        _p2a_body,
        out_type=[
            jax.ShapeDtypeStruct((EP,), jnp.float32),
            jax.ShapeDtypeStruct((EP,), jnp.float32),
        ],
        mesh=_mesh,
        compiler_params=_sc_params,
        scratch_types=[
            pltpu.VMEM((C,), jnp.int32),
            pltpu.VMEM((C,), jnp.float32),
            pltpu.VMEM((C,), jnp.float32),
            pltpu.VMEM((C,), jnp.float32),
            pltpu.VMEM((C,), jnp.float32),
            pltpu.SemaphoreType.DMA,
        ],
    )(ev, dste, s0, s1, att_in)


def _p2b_scale(idx_d, alb, lidx, vrows, lo):
    for g in range(C2 // 16):
        sl = pl.ds(g * 16, 16)
        dg = idx_d[sl]
        inr = (dg >= lo) & (dg < lo + RNG)
        alb[sl] = jnp.where(inr, alb[sl], 0.0)
        lidx[sl] = lax.rem(dg, RNG)

    def quad(qq, _):
        for u in range(4):
            e = qq * 4 + u
            ab = plsc.load_gather(alb, [jnp.full((16,), e, jnp.int32)])
            for t in range(DP // 16):
                sl = pl.ds(t * 16, 16)
                vrows[e, sl] = ab * vrows[e, sl]
        return 0

    lax.fori_loop(0, C2 // 4, quad, 0)


def _p2b_body(vp, srce, dste, alpha, agg_out,
              idx_s0, idx_d0, alb0, lidx0, vrows0,
              idx_s1, idx_d1, alb1, lidx1, vrows1,
              flushb, shacc, sem0, sem1):
    cid = lax.axis_index("c")
    sid = lax.axis_index("s")

    for rp in range(2):
        lo = (cid * 2 + rp) * RNG

        def zf16(i, _):
            r = i // (DP // 16)
            t = i % (DP // 16)
            flushb[r, pl.ds(t * 16, 16)] = jnp.zeros((16,), jnp.float32)
            return 0

        lax.fori_loop(0, FL * (DP // 16), zf16, 0)
        for h in range(NFL):
            pltpu.sync_copy(
                flushb, shacc.at[pl.ds(sid * NFL * FL + h * FL, FL)])
        plsc.subcore_barrier()

        def pair(jj, _):
            b0 = sid * ESPAN2 + (2 * jj) * C2
            b1 = b0 + C2
            pltpu.sync_copy(srce.at[pl.ds(b0, C2)], idx_s0)
            pltpu.sync_copy(dste.at[pl.ds(b0, C2)], idx_d0)
            pltpu.sync_copy(alpha.at[pl.ds(b0, C2)], alb0)
            h0 = pltpu.async_copy(vp.at[idx_s0], vrows0, sem0)
            pltpu.sync_copy(srce.at[pl.ds(b1, C2)], idx_s1)
            pltpu.sync_copy(dste.at[pl.ds(b1, C2)], idx_d1)
            pltpu.sync_copy(alpha.at[pl.ds(b1, C2)], alb1)
            h1 = pltpu.async_copy(vp.at[idx_s1], vrows1, sem1)
            h0.wait()
            _p2b_scale(idx_d0, alb0, lidx0, vrows0, lo)
            pltpu.sync_copy(vrows0, shacc.at[lidx0], add=True)
            h1.wait()
            _p2b_scale(idx_d1, alb1, lidx1, vrows1, lo)
            pltpu.sync_copy(vrows1, shacc.at[lidx1], add=True)
            return 0

        lax.fori_loop(0, NCHUNK2 // 2, pair, 0)
        plsc.subcore_barrier()

        for h in range(NFL):
            roff = sid * NFL * FL + h * FL
            pltpu.sync_copy(shacc.at[pl.ds(roff, FL)], flushb)
            pltpu.sync_copy(flushb, agg_out.at[pl.ds(lo + roff, FL)])
        plsc.subcore_barrier()


@jax.jit
def _p2b(vp, srce, dste, alpha):
    return pl.kernel(
        _p2b_body,
        out_type=jax.ShapeDtypeStruct((NP, DP), jnp.float32),
        mesh=_mesh,
        compiler_params=_sc_params,
        scratch_types=(
            [pltpu.VMEM((C2,), jnp.int32),
             pltpu.VMEM((C2,), jnp.int32),
             pltpu.VMEM((C2,), jnp.float32),
             pltpu.VMEM((C2,), jnp.int32),
             pltpu.VMEM((C2, DP), jnp.float32)] * 2
            + [pltpu.VMEM((FL, DP), jnp.float32),
               pltpu.VMEM_SHARED((RNG, DP), jnp.float32),
               pltpu.SemaphoreType.DMA,
               pltpu.SemaphoreType.DMA]
        ),
    )(vp, srce, dste, alpha)


def _dense1_body(x_ref, w_ref, b_ref, q_ref, k_ref, v_ref, s_ref):
    acc = jnp.dot(x_ref[...], w_ref[...],
                  preferred_element_type=jnp.float32)
    acc = acc + b_ref[0:1, :]
    q_ref[...] = acc[:, 0:DP]
    k_ref[...] = acc[:, DP:2 * DP]
    v_ref[...] = acc[:, 2 * DP:3 * DP]
    s_ref[...] = acc[:, 3 * DP:4 * DP]


@jax.jit
def _dense1(xp, w4, b4):
    blk = lambda i: (i, 0)
    full = lambda i: (0, 0)
    return pl.pallas_call(
        _dense1_body,
        grid=(GRID,),
        in_specs=[
            pl.BlockSpec((BM, DP), blk),
            pl.BlockSpec((DP, 4 * DP), full),
            pl.BlockSpec((8, 4 * DP), full),
        ],
        out_specs=[pl.BlockSpec((BM, DP), blk)] * 4,
        out_shape=[jax.ShapeDtypeStruct((NP, DP), jnp.float32)] * 4,
    )(xp, w4, b4)


def _stats_body(agg_ref, skip_ref, y_ref, sum_ref, ssq_ref):
    i = pl.program_id(0)
    rows = i * BM + lax.broadcasted_iota(jnp.int32, (BM, DP), 0)
    y = agg_ref[...] + skip_ref[...]
    y = jnp.where(rows < N, y, 0.0)
    y_ref[...] = y
    ps = jnp.zeros((8, DP), jnp.float32)
    pq = jnp.zeros((8, DP), jnp.float32)
    for j in range(BM // 8):
        blkv = y[j * 8:(j + 1) * 8, :]
        ps = ps + blkv
        pq = pq + blkv * blkv

    @pl.when(i == 0)
    def _():
        sum_ref[...] = ps
        ssq_ref[...] = pq

    @pl.when(i > 0)
    def _():
        sum_ref[...] = sum_ref[...] + ps
        ssq_ref[...] = ssq_ref[...] + pq


@jax.jit
def _stats(agg, skip):
    blk = lambda i: (i, 0)
    acc = lambda i: (0, 0)
    return pl.pallas_call(
        _stats_body,
        grid=(GRID,),
        in_specs=[pl.BlockSpec((BM, DP), blk), pl.BlockSpec((BM, DP), blk)],
        out_specs=[pl.BlockSpec((BM, DP), blk),
                   pl.BlockSpec((8, DP), acc), pl.BlockSpec((8, DP), acc)],
        out_shape=[jax.ShapeDtypeStruct((NP, DP), jnp.float32),
                   jax.ShapeDtypeStruct((8, DP), jnp.float32),
                   jax.ShapeDtypeStruct((8, DP), jnp.float32)],
    )(agg, skip)


def _apply_body(y_ref, sum_ref, ssq_ref, g_ref, b_ref, x0_ref, out_ref,
                *, residual):
    i = pl.program_id(0)
    colsum = jnp.sum(sum_ref[...], axis=0, keepdims=True)
    colssq = jnp.sum(ssq_ref[...], axis=0, keepdims=True)
    mean = colsum / N
    var = colssq / N - mean * mean
    scale = g_ref[0:1, :] * lax.rsqrt(var + 1e-5)
    out = (y_ref[...] - mean) * scale + b_ref[0:1, :]
    if residual:
        out = out + x0_ref[...]
    out = jnp.maximum(out, 0.0)
    rows = i * BM + lax.broadcasted_iota(jnp.int32, (BM, DP), 0)
    out_ref[...] = jnp.where(rows < N, out, 0.0)


@functools.partial(jax.jit, static_argnames=("residual",))
def _apply(y, s1, s2, g, b, x0, residual):
    blk = lambda i: (i, 0)
    full = lambda i: (0, 0)
    return pl.pallas_call(
        functools.partial(_apply_body, residual=residual),
        grid=(GRID,),
        in_specs=[pl.BlockSpec((BM, DP), blk),
                  pl.BlockSpec((8, DP), full), pl.BlockSpec((8, DP), full),
                  pl.BlockSpec((8, DP), full), pl.BlockSpec((8, DP), full),
                  pl.BlockSpec((BM, DP), blk)],
        out_specs=pl.BlockSpec((BM, DP), blk),
        out_shape=jax.ShapeDtypeStruct((NP, DP), jnp.float32),
    )(y, s1, s2, g, b, x0)


def _pad_edges(edges):
    src = edges[0].astype(jnp.int32)
    dst = edges[1].astype(jnp.int32)
    pad = EP - E
    sent = N + (jnp.arange(pad, dtype=jnp.int32) % NSENT)
    return jnp.concatenate([src, sent]), jnp.concatenate([dst, sent])


def _w4(p, pre):
    ws, bs = [], []
    for name in ("q", "k", "v", "skip"):
        W = p[pre + "_" + name + "_W"]
        b = p[pre + "_" + name + "_b"]
        ws.append(jnp.zeros((DP, DP), jnp.float32).at[:D, :D].set(W))
        bs.append(jnp.zeros((DP,), jnp.float32).at[:D].set(b))
    w4 = jnp.concatenate(ws, axis=1)
    b4 = jnp.tile(jnp.concatenate(bs)[None, :], (8, 1))
    return w4, b4


def _tile8(vec):
    v = jnp.zeros((DP,), jnp.float32).at[:D].set(vec)
    return jnp.tile(v[None, :], (8, 1))


def _conv(xp, srce, dste, att_in, p, pre):
    w4, b4 = _w4(p, pre)
    q, k, v, skip = _dense1(xp, w4, b4)
    kmax2 = jnp.max(jnp.sum(k * k, axis=1))
    mshift = (jnp.sum(q * q, axis=1) + kmax2) * (0.5 * INVSQ)
    ev, s0, s1 = _p1(q, k, srce, dste, mshift)
    alpha, att_out = _p2a(ev, dste, s0, s1, att_in)
    agg = _p2b(v, srce, dste, alpha)
    return agg, skip, att_out


def _block(x0, srce, dste, att_in, p):
    agg, skip, att1 = _conv(x0, srce, dste, att_in, p, "conv1")
    y, s1, s2 = _stats(agg, skip)
    x = _apply(y, s1, s2, _tile8(p["bn1_gamma"]), _tile8(p["bn1_beta"]),
               x0, residual=False)
    agg, skip, att2 = _conv(x, srce, dste, att1, p, "conv2")
    y, s1, s2 = _stats(agg, skip)
    x = _apply(y, s1, s2, _tile8(p["bn2_gamma"]), _tile8(p["bn2_beta"]),
               x0, residual=True)
    return x, att2


def kernel(x, edges, params):
    srce, dste = _pad_edges(edges)
    xp = jnp.zeros((NP, DP), jnp.float32).at[:N, :D].set(x)
    att = jnp.zeros((EP,), jnp.float32)
    xp, att = _block(xp, srce, dste, att, params[0])
    xp, att = _block(xp, srce, dste, att, params[1])
    xp, att = _block(xp, srce, dste, att, params[2])
    return (xp[:N, :D], (edges, att[:E, None]))


# speedup vs baseline: 2.7991x; 1.0033x over previous
import functools

import jax
import jax.numpy as jnp
from jax import lax
from jax.experimental import pallas as pl
from jax.experimental.pallas import tpu as pltpu
from jax.experimental.pallas import tpu_sc as plsc

N = 50000
E = 800000
D = 100
DP = 128
NP = 50176
EP = 802816
NSENT = NP - N
NC = 2
NS = 16
NW = NC * NS
ESPAN = EP // NW
C = 128
NCHUNK = ESPAN // C
SUBN = NP // NS
INVSQ = 0.1
RNG = NP // 4
ESPAN2 = EP // NS
C2 = 64
NCHUNK2 = ESPAN2 // C2
FL = 56
NFL = 784 // FL
BM = 512
GRID = NP // BM

_mesh = plsc.VectorSubcoreMesh(core_axis_name="c", subcore_axis_name="s")
_sc_params = pltpu.CompilerParams(needs_layout_passes=False)


def _p1_compute(qrows, krows, mrow, evb):
    def group(g, _):
        lanes = g * 16 + lax.iota(jnp.int32, 16)
        acc = jnp.zeros((16,), jnp.float32)
        for d in range(D):
            dd = jnp.full((16,), d, jnp.int32)
            qv = plsc.load_gather(qrows, [lanes, dd])
            kv = plsc.load_gather(krows, [lanes, dd])
            acc = acc + qv * kv
        mg = mrow[pl.ds(g * 16, 16)]
        evb[pl.ds(g * 16, 16)] = jnp.exp(acc * INVSQ - mg)
        return 0

    lax.fori_loop(0, C // 16, group, 0)


def _p1_body(qp, kp, srce, dste, msh, ev_out, s0_out, s1_out,
             idx_s0, idx_d0, q0, k0, m0, ev0,
             idx_s1, idx_d1, q1, k1, m1, ev1,
             zb, shacc, sem0, sem1):
    cid = lax.axis_index("c")
    sid = lax.axis_index("s")
    wid = sid * NC + cid

    def zeros16(i, _):
        zb[pl.ds(i * 16, 16)] = jnp.zeros((16,), jnp.float32)
        return 0

    lax.fori_loop(0, SUBN // 16, zeros16, 0)
    pltpu.sync_copy(zb, shacc.at[pl.ds(sid * SUBN, SUBN)])
    plsc.subcore_barrier()

    def pair(jj, _):
        b0 = wid * ESPAN + (2 * jj) * C
        b1 = b0 + C
        pltpu.sync_copy(srce.at[pl.ds(b0, C)], idx_s0)
        pltpu.sync_copy(dste.at[pl.ds(b0, C)], idx_d0)
        h0q = pltpu.async_copy(qp.at[idx_d0], q0, sem0)
        h0k = pltpu.async_copy(kp.at[idx_s0], k0, sem0)
        h0m = pltpu.async_copy(msh.at[idx_d0], m0, sem0)
        pltpu.sync_copy(srce.at[pl.ds(b1, C)], idx_s1)
        pltpu.sync_copy(dste.at[pl.ds(b1, C)], idx_d1)
        h1q = pltpu.async_copy(qp.at[idx_d1], q1, sem1)
        h1k = pltpu.async_copy(kp.at[idx_s1], k1, sem1)
        h1m = pltpu.async_copy(msh.at[idx_d1], m1, sem1)
        h0q.wait(); h0k.wait(); h0m.wait()
        _p1_compute(q0, k0, m0, ev0)
        pltpu.sync_copy(ev0, ev_out.at[pl.ds(b0, C)])
        pltpu.sync_copy(ev0, shacc.at[idx_d0], add=True)
        h1q.wait(); h1k.wait(); h1m.wait()
        _p1_compute(q1, k1, m1, ev1)
        pltpu.sync_copy(ev1, ev_out.at[pl.ds(b1, C)])
        pltpu.sync_copy(ev1, shacc.at[idx_d1], add=True)
        return 0

    lax.fori_loop(0, NCHUNK // 2, pair, 0)

    plsc.subcore_barrier()
    pltpu.sync_copy(shacc.at[pl.ds(sid * SUBN, SUBN)], zb)

    @pl.when(cid == 0)
    def _():
        pltpu.sync_copy(zb, s0_out.at[pl.ds(sid * SUBN, SUBN)])

    @pl.when(cid == 1)
    def _():
        pltpu.sync_copy(zb, s1_out.at[pl.ds(sid * SUBN, SUBN)])


@jax.jit
def _p1(qp, kp, srce, dste, msh):
    return pl.kernel(
        _p1_body,
        out_type=[
            jax.ShapeDtypeStruct((EP,), jnp.float32),
            jax.ShapeDtypeStruct((NP,), jnp.float32),
            jax.ShapeDtypeStruct((NP,), jnp.float32),
        ],
        mesh=_mesh,
        compiler_params=_sc_params,
        scratch_types=(
            [pltpu.VMEM((C,), jnp.int32),
             pltpu.VMEM((C,), jnp.int32),
             pltpu.VMEM((C, DP), jnp.float32),
             pltpu.VMEM((C, DP), jnp.float32),
             pltpu.VMEM((C,), jnp.float32),
             pltpu.VMEM((C,), jnp.float32)] * 2
            + [pltpu.VMEM((SUBN,), jnp.float32),
               pltpu.VMEM_SHARED((NP,), jnp.float32),
               pltpu.SemaphoreType.DMA,
               pltpu.SemaphoreType.DMA]
        ),
    )(qp, kp, srce, dste, msh)


def _p2a_body(ev, dste, s0, s1, att_in, alpha_out, att_out,
              idx_d, evb, attb, s0b, s1b, sem):
    cid = lax.axis_index("c")
    sid = lax.axis_index("s")
    wid = sid * NC + cid

    def chunk(j, _):
        base = wid * ESPAN + j * C
        pltpu.sync_copy(dste.at[pl.ds(base, C)], idx_d)
        h0 = pltpu.async_copy(s0.at[idx_d], s0b, sem)
        h1 = pltpu.async_copy(s1.at[idx_d], s1b, sem)
        pltpu.sync_copy(ev.at[pl.ds(base, C)], evb)
        pltpu.sync_copy(att_in.at[pl.ds(base, C)], attb)
        h0.wait()
        h1.wait()

        def group(g, _):
            sl = pl.ds(g * 16, 16)
            al = evb[sl] / (s0b[sl] + s1b[sl])
            attb[sl] = attb[sl] + al
            evb[sl] = al
            return 0

        lax.fori_loop(0, C // 16, group, 0)
        pltpu.sync_copy(evb, alpha_out.at[pl.ds(base, C)])
        pltpu.sync_copy(attb, att_out.at[pl.ds(base, C)])
        return 0

    lax.fori_loop(0, NCHUNK, chunk, 0)


@jax.jit
def _p2a(ev, dste, s0, s1, att_in):
    return pl.kernel(
        _p2a_body,
        out_type=[
            jax.ShapeDtypeStruct((EP,), jnp.float32),
            jax.ShapeDtypeStruct((EP,), jnp.float32),
        ],
        mesh=_mesh,
        compiler_params=_sc_params,
        scratch_types=[
            pltpu.VMEM((C,), jnp.int32),
            pltpu.VMEM((C,), jnp.float32),
            pltpu.VMEM((C,), jnp.float32),
            pltpu.VMEM((C,), jnp.float32),
            pltpu.VMEM((C,), jnp.float32),
            pltpu.SemaphoreType.DMA,
        ],
    )(ev, dste, s0, s1, att_in)


def _p2b_scale(idx_d, alb, lidx, vrows, lo):
    for g in range(C2 // 16):
        sl = pl.ds(g * 16, 16)
        dg = idx_d[sl]
        inr = (dg >= lo) & (dg < lo + RNG)
        alb[sl] = jnp.where(inr, alb[sl], 0.0)
        lidx[sl] = lax.rem(dg, RNG)

    def quad(qq, _):
        for u in range(4):
            e = qq * 4 + u
            ab = plsc.load_gather(alb, [jnp.full((16,), e, jnp.int32)])
            for t in range(DP // 16):
                sl = pl.ds(t * 16, 16)
                vrows[e, sl] = ab * vrows[e, sl]
        return 0

    lax.fori_loop(0, C2 // 4, quad, 0)


def _p2b_body(vp, srce, dste, alpha, agg_out,
              idx_s0, idx_d0, alb0, lidx0, vrows0,
              idx_s1, idx_d1, alb1, lidx1, vrows1,
              flushb, shacc, sem0, sem1):
    cid = lax.axis_index("c")
    sid = lax.axis_index("s")

    for rp in range(2):
        lo = (cid * 2 + rp) * RNG

        def zf16(i, _):
            r = i // (DP // 16)
            t = i % (DP // 16)
            flushb[r, pl.ds(t * 16, 16)] = jnp.zeros((16,), jnp.float32)
            return 0

        lax.fori_loop(0, FL * (DP // 16), zf16, 0)
        for h in range(NFL):
            pltpu.sync_copy(
                flushb, shacc.at[pl.ds(sid * NFL * FL + h * FL, FL)])
        plsc.subcore_barrier()

        def pair(jj, _):
            b0 = sid * ESPAN2 + (2 * jj) * C2
            b1 = b0 + C2
            pltpu.sync_copy(srce.at[pl.ds(b0, C2)], idx_s0)
            pltpu.sync_copy(dste.at[pl.ds(b0, C2)], idx_d0)
            pltpu.sync_copy(alpha.at[pl.ds(b0, C2)], alb0)
            h0 = pltpu.async_copy(vp.at[idx_s0], vrows0, sem0)
            pltpu.sync_copy(srce.at[pl.ds(b1, C2)], idx_s1)
            pltpu.sync_copy(dste.at[pl.ds(b1, C2)], idx_d1)
            pltpu.sync_copy(alpha.at[pl.ds(b1, C2)], alb1)
            h1 = pltpu.async_copy(vp.at[idx_s1], vrows1, sem1)
            h0.wait()
            _p2b_scale(idx_d0, alb0, lidx0, vrows0, lo)
            pltpu.sync_copy(vrows0, shacc.at[lidx0], add=True)
            h1.wait()
            _p2b_scale(idx_d1, alb1, lidx1, vrows1, lo)
            pltpu.sync_copy(vrows1, shacc.at[lidx1], add=True)
            return 0

        lax.fori_loop(0, NCHUNK2 // 2, pair, 0)
        plsc.subcore_barrier()

        for h in range(NFL):
            roff = sid * NFL * FL + h * FL
            pltpu.sync_copy(shacc.at[pl.ds(roff, FL)], flushb)
            pltpu.sync_copy(flushb, agg_out.at[pl.ds(lo + roff, FL)])
        plsc.subcore_barrier()


@jax.jit
def _p2b(vp, srce, dste, alpha):
    return pl.kernel(
        _p2b_body,
        out_type=jax.ShapeDtypeStruct((NP, DP), jnp.float32),
        mesh=_mesh,
        compiler_params=_sc_params,
        scratch_types=(
            [pltpu.VMEM((C2,), jnp.int32),
             pltpu.VMEM((C2,), jnp.int32),
             pltpu.VMEM((C2,), jnp.float32),
             pltpu.VMEM((C2,), jnp.int32),
             pltpu.VMEM((C2, DP), jnp.float32)] * 2
            + [pltpu.VMEM((FL, DP), jnp.float32),
               pltpu.VMEM_SHARED((RNG, DP), jnp.float32),
               pltpu.SemaphoreType.DMA,
               pltpu.SemaphoreType.DMA]
        ),
    )(vp, srce, dste, alpha)


def _dense1_body(x_ref, w_ref, b_ref, q_ref, k_ref, v_ref, s_ref):
    acc = jnp.dot(x_ref[...], w_ref[...],
                  preferred_element_type=jnp.float32)
    acc = acc + b_ref[0:1, :]
    q_ref[...] = acc[:, 0:DP]
    k_ref[...] = acc[:, DP:2 * DP]
    v_ref[...] = acc[:, 2 * DP:3 * DP]
    s_ref[...] = acc[:, 3 * DP:4 * DP]


@jax.jit
def _dense1(xp, w4, b4):
    blk = lambda i: (i, 0)
    full = lambda i: (0, 0)
    return pl.pallas_call(
        _dense1_body,
        grid=(GRID,),
        in_specs=[
            pl.BlockSpec((BM, DP), blk),
            pl.BlockSpec((DP, 4 * DP), full),
            pl.BlockSpec((8, 4 * DP), full),
        ],
        out_specs=[pl.BlockSpec((BM, DP), blk)] * 4,
        out_shape=[jax.ShapeDtypeStruct((NP, DP), jnp.float32)] * 4,
    )(xp, w4, b4)


def _stats_body(agg_ref, skip_ref, y_ref, sum_ref, ssq_ref):
    i = pl.program_id(0)
    rows = i * BM + lax.broadcasted_iota(jnp.int32, (BM, DP), 0)
    y = agg_ref[...] + skip_ref[...]
    y = jnp.where(rows < N, y, 0.0)
    y_ref[...] = y
    ps = jnp.zeros((8, DP), jnp.float32)
    pq = jnp.zeros((8, DP), jnp.float32)
    for j in range(BM // 8):
        blkv = y[j * 8:(j + 1) * 8, :]
        ps = ps + blkv
        pq = pq + blkv * blkv

    @pl.when(i == 0)
    def _():
        sum_ref[...] = ps
        ssq_ref[...] = pq

    @pl.when(i > 0)
    def _():
        sum_ref[...] = sum_ref[...] + ps
        ssq_ref[...] = ssq_ref[...] + pq


@jax.jit
def _stats(agg, skip):
    blk = lambda i: (i, 0)
    acc = lambda i: (0, 0)
    return pl.pallas_call(
        _stats_body,
        grid=(GRID,),
        in_specs=[pl.BlockSpec((BM, DP), blk), pl.BlockSpec((BM, DP), blk)],
        out_specs=[pl.BlockSpec((BM, DP), blk),
                   pl.BlockSpec((8, DP), acc), pl.BlockSpec((8, DP), acc)],
        out_shape=[jax.ShapeDtypeStruct((NP, DP), jnp.float32),
                   jax.ShapeDtypeStruct((8, DP), jnp.float32),
                   jax.ShapeDtypeStruct((8, DP), jnp.float32)],
    )(agg, skip)


def _apply_body(y_ref, sum_ref, ssq_ref, g_ref, b_ref, x0_ref, out_ref,
                *, residual):
    i = pl.program_id(0)
    colsum = jnp.sum(sum_ref[...], axis=0, keepdims=True)
    colssq = jnp.sum(ssq_ref[...], axis=0, keepdims=True)
    mean = colsum / N
    var = colssq / N - mean * mean
    scale = g_ref[0:1, :] * lax.rsqrt(var + 1e-5)
    out = (y_ref[...] - mean) * scale + b_ref[0:1, :]
    if residual:
        out = out + x0_ref[...]
    out = jnp.maximum(out, 0.0)
    rows = i * BM + lax.broadcasted_iota(jnp.int32, (BM, DP), 0)
    out_ref[...] = jnp.where(rows < N, out, 0.0)


@functools.partial(jax.jit, static_argnames=("residual",))
def _apply(y, s1, s2, g, b, x0, residual):
    blk = lambda i: (i, 0)
    full = lambda i: (0, 0)
    return pl.pallas_call(
        functools.partial(_apply_body, residual=residual),
        grid=(GRID,),
        in_specs=[pl.BlockSpec((BM, DP), blk),
                  pl.BlockSpec((8, DP), full), pl.BlockSpec((8, DP), full),
                  pl.BlockSpec((8, DP), full), pl.BlockSpec((8, DP), full),
                  pl.BlockSpec((BM, DP), blk)],
        out_specs=pl.BlockSpec((BM, DP), blk),
        out_shape=jax.ShapeDtypeStruct((NP, DP), jnp.float32),
    )(y, s1, s2, g, b, x0)


def _pad_edges(edges):
    src = edges[0].astype(jnp.int32)
    dst = edges[1].astype(jnp.int32)
    pad = EP - E
    sent = N + (jnp.arange(pad, dtype=jnp.int32) % NSENT)
    return jnp.concatenate([src, sent]), jnp.concatenate([dst, sent])


def _w4(p, pre):
    ws, bs = [], []
    for name in ("q", "k", "v", "skip"):
        W = p[pre + "_" + name + "_W"]
        b = p[pre + "_" + name + "_b"]
        ws.append(jnp.zeros((DP, DP), jnp.float32).at[:D, :D].set(W))
        bs.append(jnp.zeros((DP,), jnp.float32).at[:D].set(b))
    w4 = jnp.concatenate(ws, axis=1)
    b4 = jnp.tile(jnp.concatenate(bs)[None, :], (8, 1))
    return w4, b4


def _tile8(vec):
    v = jnp.zeros((DP,), jnp.float32).at[:D].set(vec)
    return jnp.tile(v[None, :], (8, 1))


def _conv(xp, srce, dste, att_in, p, pre):
    w4, b4 = _w4(p, pre)
    q, k, v, skip = _dense1(xp, w4, b4)
    kmax2 = jnp.max(jnp.sum(k * k, axis=1))
    mshift = (jnp.sum(q * q, axis=1) + kmax2) * (0.5 * INVSQ)
    ev, s0, s1 = _p1(q, k, srce, dste, mshift)
    alpha, att_out = _p2a(ev, dste, s0, s1, att_in)
    agg = _p2b(v, srce, dste, alpha)
    return agg, skip, att_out


def _block(x0, srce, dste, att_in, p):
    agg, skip, att1 = _conv(x0, srce, dste, att_in, p, "conv1")
    y, s1, s2 = _stats(agg, skip)
    x = _apply(y, s1, s2, _tile8(p["bn1_gamma"]), _tile8(p["bn1_beta"]),
               x0, residual=False)
    agg, skip, att2 = _conv(x, srce, dste, att1, p, "conv2")
    y, s1, s2 = _stats(agg, skip)
    x = _apply(y, s1, s2, _tile8(p["bn2_gamma"]), _tile8(p["bn2_beta"]),
               x0, residual=True)
    return x, att2


def kernel(x, edges, params):
    srce, dste = _pad_edges(edges)
    xp = jnp.zeros((NP, DP), jnp.float32).at[:N, :D].set(x)
    att = jnp.zeros((EP,), jnp.float32)
    xp, att = _block(xp, srce, dste, att, params[0])
    xp, att = _block(xp, srce, dste, att, params[1])
    xp, att = _block(xp, srce, dste, att, params[2])
    return (xp[:N, :D], (edges, att[:E, None]))
